# baseline (speedup 1.0000x reference)
import functools

import jax
import jax.numpy as jnp
from jax import lax
from jax.experimental import pallas as pl
from jax.experimental.pallas import tpu as pltpu
from jax.experimental.pallas import tpu_sc as plsc

H = 512
CH = 128
BO = 256
BM = 256
BMM = 512
NW = 32
R_E = 8
R_N = 7


def _rup(x, m):
    return (x + m - 1) // m * m


def _sc_gather(table, idx):
    b_total = idx.shape[0]
    d = table.shape[1]
    bw = b_total // NW
    nch = bw // CH
    mesh = plsc.VectorSubcoreMesh(core_axis_name="c", subcore_axis_name="s")

    @functools.partial(
        pl.kernel,
        mesh=mesh,
        out_type=jax.ShapeDtypeStruct((b_total, d), jnp.float32),
        scratch_types=[
            pltpu.VMEM((CH,), jnp.int32),
            pltpu.VMEM((CH, d), jnp.float32),
            pltpu.SemaphoreType.DMA,
        ],
    )
    def k(table_hbm, idx_hbm, out_hbm, idx_v, rows_v, sem):
        wid = lax.axis_index("s") * 2 + lax.axis_index("c")
        base = wid * bw

        def body(j, carry):
            off = base + j * CH
            pltpu.sync_copy(idx_hbm.at[pl.ds(off, CH)], idx_v)
            pltpu.async_copy(table_hbm.at[idx_v], rows_v, sem).wait()
            pltpu.sync_copy(rows_v, out_hbm.at[pl.ds(off, CH)])
            return carry

        lax.fori_loop(0, nch, body, 0)

    return k(table, idx)


def _mm(x, w, b):
    m, kdim = x.shape
    n2 = w.shape[1]
    nblk = (m + BMM - 1) // BMM

    def body(x_ref, w_ref, b_ref, out_ref):
        out_ref[...] = (
            jnp.dot(x_ref[...], w_ref[...], preferred_element_type=jnp.float32)
            + b_ref[...]
        )

    return pl.pallas_call(
        body,
        grid=(nblk,),
        in_specs=[
            pl.BlockSpec((BMM, kdim), lambda c: (c, 0)),
            pl.BlockSpec((kdim, n2), lambda c: (0, 0)),
            pl.BlockSpec((1, n2), lambda c: (0, 0)),
        ],
        out_specs=pl.BlockSpec((BMM, n2), lambda c: (c, 0)),
        out_shape=jax.ShapeDtypeStruct((m, n2), jnp.float32),
    )(x, w, b.reshape(1, n2))


def _bdmm(ge, wstack, relmap):
    l1, din = ge.shape
    nblk = l1 // BM

    grid_spec = pltpu.PrefetchScalarGridSpec(
        num_scalar_prefetch=1,
        grid=(nblk,),
        in_specs=[
            pl.BlockSpec((BM, din), lambda c, rm: (c, 0)),
            pl.BlockSpec((1, din, H), lambda c, rm: (rm[c], 0, 0)),
        ],
        out_specs=pl.BlockSpec((BM, H), lambda c, rm: (c, 0)),
    )

    def body(rm_ref, ge_ref, w_ref, out_ref):
        out_ref[...] = jnp.dot(
            ge_ref[...], w_ref[0], preferred_element_type=jnp.float32
        )

    return pl.pallas_call(
        body,
        grid_spec=grid_spec,
        out_shape=jax.ShapeDtypeStruct((l1, H), jnp.float32),
    )(relmap, ge, wstack)


def _seg_emp(msg, dl2, sw, blkmap, firstmap, lastmap, nb):
    nch = msg.shape[0] // CH

    grid_spec = pltpu.PrefetchScalarGridSpec(
        num_scalar_prefetch=3,
        grid=(nch,),
        in_specs=[
            pl.BlockSpec((CH, H), lambda c, bm, fm, lm: (c, 0)),
            pl.BlockSpec((1, 1, CH), lambda c, bm, fm, lm: (c, 0, 0)),
            pl.BlockSpec((BO, H), lambda c, bm, fm, lm: (bm[c], 0)),
        ],
        out_specs=pl.BlockSpec((BO, H), lambda c, bm, fm, lm: (bm[c], 0)),
    )

    def body(bm_ref, fm_ref, lm_ref, msg_ref, dl_ref, sw_ref, out_ref):
        c = pl.program_id(0)
        oh = (
            lax.broadcasted_iota(jnp.int32, (BO, CH), 0) == dl_ref[0]
        ).astype(jnp.float32)
        contrib = jnp.dot(oh, msg_ref[...], preferred_element_type=jnp.float32)

        @pl.when(fm_ref[c] == 1)
        def _():
            out_ref[...] = sw_ref[...] + contrib

        @pl.when(fm_ref[c] == 0)
        def _():
            out_ref[...] = out_ref[...] + contrib

        @pl.when(lm_ref[c] == 1)
        def _():
            out_ref[...] = jnp.maximum(out_ref[...], 0.0)

    return pl.pallas_call(
        body,
        grid_spec=grid_spec,
        out_shape=jax.ShapeDtypeStruct((nb * BO, H), jnp.float32),
    )(blkmap, firstmap, lastmap, msg, dl2.reshape(nch, 1, CH), sw)


def _seg_rgcn(ma, mb, dl2, init, blkmap, firstmap, nb):
    nch = ma.shape[0] // CH

    grid_spec = pltpu.PrefetchScalarGridSpec(
        num_scalar_prefetch=2,
        grid=(nch,),
        in_specs=[
            pl.BlockSpec((CH, H), lambda c, bm, fm: (c, 0)),
            pl.BlockSpec((CH, H), lambda c, bm, fm: (c, 0)),
            pl.BlockSpec((1, 1, CH), lambda c, bm, fm: (c, 0, 0)),
            pl.BlockSpec((BO, H), lambda c, bm, fm: (bm[c], 0)),
        ],
        out_specs=pl.BlockSpec((BO, H), lambda c, bm, fm: (bm[c], 0)),
    )

    def body(bm_ref, fm_ref, ma_ref, mb_ref, dl_ref, init_ref, out_ref):
        c = pl.program_id(0)
        oh = (
            lax.broadcasted_iota(jnp.int32, (BO, CH), 0) == dl_ref[0]
        ).astype(jnp.float32)
        contrib = jnp.dot(
            oh, ma_ref[...] + mb_ref[...], preferred_element_type=jnp.float32
        )

        @pl.when(fm_ref[c] == 1)
        def _():
            out_ref[...] = init_ref[...] + contrib

        @pl.when(fm_ref[c] == 0)
        def _():
            out_ref[...] = out_ref[...] + contrib

    return pl.pallas_call(
        body,
        grid_spec=grid_spec,
        out_shape=jax.ShapeDtypeStruct((nb * BO, H), jnp.float32),
    )(blkmap, firstmap, ma, mb, dl2.reshape(nch, 1, CH), init)


def _stats(y):
    n = y.shape[0]
    chs = 256
    nblk = (n + chs - 1) // chs

    def body(y_ref, out_ref):
        c = pl.program_id(0)
        rows = lax.broadcasted_iota(jnp.int32, (chs, 1), 0) + c * chs
        ym = jnp.where(rows < n, y_ref[...], 0.0)

        @pl.when(c == 0)
        def _():
            out_ref[...] = jnp.zeros((8, H), jnp.float32)

        out_ref[0:1, :] = out_ref[0:1, :] + jnp.sum(ym, axis=0, keepdims=True)
        out_ref[1:2, :] = out_ref[1:2, :] + jnp.sum(ym * ym, axis=0, keepdims=True)

    return pl.pallas_call(
        body,
        grid=(nblk,),
        in_specs=[pl.BlockSpec((chs, H), lambda c: (c, 0))],
        out_specs=pl.BlockSpec((8, H), lambda c: (0, 0)),
        out_shape=jax.ShapeDtypeStruct((8, H), jnp.float32),
    )(y)


def _bn_apply(y, st, g, beta, relu, n):
    rows = y.shape[0]
    chb = 512
    nblk = (rows + chb - 1) // chb

    def body(y_ref, s_ref, g_ref, b_ref, out_ref):
        m = s_ref[0:1, :] / n
        v = s_ref[1:2, :] / n - m * m
        inv = lax.rsqrt(v + 1e-5)
        o = (y_ref[...] - m) * inv * g_ref[...] + b_ref[...]
        if relu:
            o = jnp.maximum(o, 0.0)
        out_ref[...] = o

    return pl.pallas_call(
        body,
        grid=(nblk,),
        in_specs=[
            pl.BlockSpec((chb, H), lambda c: (c, 0)),
            pl.BlockSpec((8, H), lambda c: (0, 0)),
            pl.BlockSpec((1, H), lambda c: (0, 0)),
            pl.BlockSpec((1, H), lambda c: (0, 0)),
        ],
        out_specs=pl.BlockSpec((chb, H), lambda c: (c, 0)),
        out_shape=jax.ShapeDtypeStruct((rows, H), jnp.float32),
    )(y, st, g, beta)


def _pool(h, bids2, g):
    nchp = bids2.shape[0]
    n = h.shape[0]

    def body(h_ref, b_ref, sum_ref, cnt_ref):
        c = pl.program_id(0)
        rows = lax.broadcasted_iota(jnp.int32, (CH, 1), 0) + c * CH
        hm = jnp.where(rows < n, h_ref[...], 0.0)
        oh = (
            lax.broadcasted_iota(jnp.int32, (g, CH), 0) == b_ref[0]
        ).astype(jnp.float32)

        @pl.when(c == 0)
        def _():
            sum_ref[...] = jnp.zeros((g, H), jnp.float32)
            cnt_ref[...] = jnp.zeros((g, 128), jnp.float32)

        sum_ref[...] = sum_ref[...] + jnp.dot(
            oh, hm, preferred_element_type=jnp.float32
        )
        cnt_ref[...] = cnt_ref[...] + jnp.dot(
            oh, jnp.ones((CH, 128), jnp.float32), preferred_element_type=jnp.float32
        )

    return pl.pallas_call(
        body,
        grid=(nchp,),
        in_specs=[
            pl.BlockSpec((CH, H), lambda c: (c, 0)),
            pl.BlockSpec((1, 1, CH), lambda c: (c, 0, 0)),
        ],
        out_specs=[
            pl.BlockSpec((g, H), lambda c: (0, 0)),
            pl.BlockSpec((g, 128), lambda c: (0, 0)),
        ],
        out_shape=[
            jax.ShapeDtypeStruct((g, H), jnp.float32),
            jax.ShapeDtypeStruct((g, 128), jnp.float32),
        ],
    )(h, bids2.reshape(nchp, 1, CH))


def _mlp(sums, cnts, p1, b1, p2, b2, p3, b3):
    g = sums.shape[0]
    dout = p3.shape[1]

    def body(s_ref, c_ref, p1_ref, b1_ref, p2_ref, b2_ref, p3_ref, b3_ref, out_ref):
        cnt = jnp.maximum(c_ref[:, 0:1], 1.0)
        pooled = s_ref[...] / cnt
        z = jnp.dot(pooled, p1_ref[...], preferred_element_type=jnp.float32) + b1_ref[...]
        z = jnp.maximum(z, 0.0)
        z = jnp.dot(z, p2_ref[...], preferred_element_type=jnp.float32) + b2_ref[...]
        z = jnp.dot(z, p3_ref[...], preferred_element_type=jnp.float32) + b3_ref[...]
        out_ref[...] = z

    return pl.pallas_call(
        body,
        out_shape=jax.ShapeDtypeStruct((g, dout), jnp.float32),
    )(sums, cnts, p1, b1.reshape(1, -1), p2, b2.reshape(1, -1), p3, b3.reshape(1, -1))


def _seg_layout(dst_ids, num_rows, nb):
    ne = dst_ids.shape[0]
    order = jnp.argsort(dst_ids)
    d_s = dst_ids[order]
    blk_s = d_s // BO
    cnt_b = jnp.bincount(blk_s, length=nb)
    chunks_b = jnp.maximum(1, (cnt_b + CH - 1) // CH)
    bounds = jnp.cumsum(chunks_b)
    cstart = bounds - chunks_b
    ecum = jnp.cumsum(cnt_b) - cnt_b
    pos = jnp.arange(ne, dtype=jnp.int32) + (cstart[blk_s] * CH - ecum[blk_s]).astype(jnp.int32)
    length = _rup(ne + nb * CH, NW * CH)
    nch = length // CH
    dloc = jnp.full((length,), -1, jnp.int32).at[pos].set(
        (d_s - blk_s * BO).astype(jnp.int32))
    blkmap = jnp.minimum(
        jnp.searchsorted(bounds, jnp.arange(nch), side="right"), nb - 1
    ).astype(jnp.int32)
    firstmap = jnp.zeros((nch,), jnp.int32).at[cstart].set(1)
    lastmap = jnp.zeros((nch,), jnp.int32).at[bounds - 1].set(1)
    return order, pos, length, blkmap, firstmap, lastmap, dloc


def kernel(x, edge_attr, params, edge_index, edge_type, edge_message_index,
           edge_message_relation, batch):
    n = x.shape[0]
    e_cnt = edge_index.shape[1]
    e2_cnt = edge_message_index.shape[1]
    g = 32
    p = params

    src = edge_index[0].astype(jnp.int32)
    dst = edge_index[1].astype(jnp.int32)
    esrc = edge_message_index[0].astype(jnp.int32)
    edst = edge_message_index[1].astype(jnp.int32)
    erel = edge_message_relation.astype(jnp.int32)
    etype = edge_type.astype(jnp.int32)

    order1 = jnp.argsort(erel)
    rel_s = erel[order1]
    cnt_r = jnp.bincount(erel, length=R_E)
    pad_r = (cnt_r + BM - 1) // BM * BM - cnt_r
    padoff = (jnp.cumsum(pad_r) - pad_r).astype(jnp.int32)
    pos1 = jnp.arange(e2_cnt, dtype=jnp.int32) + padoff[rel_s]
    l1 = _rup(e2_cnt + R_E * BM, NW * CH)
    gidx1 = jnp.zeros((l1,), jnp.int32).at[pos1].set(esrc[order1])
    relmap = jnp.zeros((l1 // BM,), jnp.int32).at[pos1 // BM].set(rel_s)
    pos1_orig = jnp.zeros((e2_cnt,), jnp.int32).at[order1].set(pos1)

    nbe = (e_cnt + BO - 1) // BO
    order3, pos3, l3, blkmap3, first3, last3, dl3 = _seg_layout(edst, e_cnt, nbe)
    idx2 = jnp.zeros((l3,), jnp.int32).at[pos3].set(pos1_orig[order3])

    nbn = (n + BO - 1) // BO
    order3n, pos3n, l3n, blkmapn, firstn, _lastn, dl3n = _seg_layout(dst, n, nbn)
    idx_a = jnp.zeros((l3n,), jnp.int32).at[pos3n].set(
        (src * R_N + etype)[order3n])
    idx_b = jnp.zeros((l3n,), jnp.int32).at[pos3n].set(order3n)

    nchp = (n + CH - 1) // CH
    bids2 = jnp.full((nchp * CH,), g + 1, jnp.int32).at[
        jnp.arange(n, dtype=jnp.int32)].set(batch.astype(jnp.int32)).reshape(nchp, CH)

    zeros_h = jnp.zeros((1, H), jnp.float32)
    bns = ["bn1", "bn2", "bn3", "bn3", "bn3", "bn2"]

    e = edge_attr
    h = x
    for i in range(6):
        li = i + 1
        w_emp = p["emp%d_W" % li]
        ws = p["emp%d_Ws" % li]
        b_emp = p["emp%d_b" % li]
        din = e.shape[1]
        dpad = _rup(din, 128)
        if dpad != din:
            ep = jnp.pad(e, ((0, 0), (0, dpad - din)))
            wp = jnp.pad(w_emp, ((0, 0), (0, dpad - din), (0, 0)))
            wsp = jnp.pad(ws, ((0, dpad - din), (0, 0)))
        else:
            ep, wp, wsp = e, w_emp, ws

        ge = _sc_gather(ep, gidx1)
        msg1 = _bdmm(ge, wp, relmap)
        msg3 = _sc_gather(msg1, idx2)
        sw = _mm(ep, wsp, b_emp)
        sw_pad = jnp.pad(sw, ((0, nbe * BO - e_cnt), (0, 0)))
        e = _seg_emp(msg3, dl3, sw_pad, blkmap3, first3, last3, nbe)[:e_cnt]

        w_conv = p["conv%d_W" % li]
        wr = p["conv%d_Wr" % li]
        we = p["conv%d_We" % li]
        b_conv = p["conv%d_b" % li]
        dinh = h.shape[1]
        hpad = _rup(dinh, 32)
        if hpad != dinh:
            hp = jnp.pad(h, ((0, 0), (0, hpad - dinh)))
            wcp = jnp.pad(w_conv, ((0, 0), (0, hpad - dinh), (0, 0)))
            wrp = jnp.pad(wr, ((0, hpad - dinh), (0, 0)))
        else:
            hp, wcp, wrp = h, w_conv, wr

        hw2 = jnp.stack(
            [_mm(hp, wcp[r], zeros_h) for r in range(R_N)], axis=1
        ).reshape(R_N * n, H)
        ewe = _mm(e, we, zeros_h)
        ma = _sc_gather(hw2, idx_a)
        mb = _sc_gather(ewe, idx_b)
        hwr = _mm(hp, wrp, b_conv)
        hwr_pad = jnp.pad(hwr, ((0, nbn * BO - n), (0, 0)))
        y = _seg_rgcn(ma, mb, dl3n, hwr_pad, blkmapn, firstn, nbn)[:n]

        st = _stats(y)
        gg = p[bns[i] + "_g"].reshape(1, H)
        bb = p[bns[i] + "_beta"].reshape(1, H)
        h = _bn_apply(y, st, gg, bb, relu=(i < 5), n=float(n))

    sums, cnts = _pool(h, bids2, g)

    dmid = p["P2"].shape[1]
    dpad2 = _rup(dmid, 128)
    p2p = jnp.pad(p["P2"], ((0, 0), (0, dpad2 - dmid)))
    pb2p = jnp.pad(p["pb2"], (0, dpad2 - dmid))
    p3p = jnp.pad(p["P3"], ((0, dpad2 - dmid), (0, dpad2 - dmid)))
    pb3p = jnp.pad(p["pb3"], (0, dpad2 - dmid))
    z = _mlp(sums, cnts, p["P1"], p["pb1"], p2p, pb2p, p3p, pb3p)
    return z[:, :dmid]


# speedup vs baseline: 1.0026x; 1.0026x over previous
import functools

import jax
import jax.numpy as jnp
from jax import lax
from jax.experimental import pallas as pl
from jax.experimental.pallas import tpu as pltpu
from jax.experimental.pallas import tpu_sc as plsc

H = 512
CH = 128
BO = 256
BM = 256
BMM = 512
NW = 32
R_E = 8
R_N = 7


def _rup(x, m):
    return (x + m - 1) // m * m


CHG = 64


def _sc_gather(table, idx):
    b_total = idx.shape[0]
    d = table.shape[1]
    bw = b_total // NW
    nch = bw // CHG
    npair = nch // 2
    mesh = plsc.VectorSubcoreMesh(core_axis_name="c", subcore_axis_name="s")

    @functools.partial(
        pl.kernel,
        mesh=mesh,
        out_type=jax.ShapeDtypeStruct((b_total, d), jnp.float32),
        scratch_types=[
            pltpu.VMEM((2, CHG), jnp.int32),
            pltpu.VMEM((2, CHG, d), jnp.float32),
            pltpu.SemaphoreType.DMA,
            pltpu.SemaphoreType.DMA,
        ],
    )
    def k(table_hbm, idx_hbm, out_hbm, idx_v, rows_v, sem0, sem1):
        wid = lax.axis_index("s") * 2 + lax.axis_index("c")
        base = wid * bw
        sems = (sem0, sem1)

        def body(jp, carry):
            off = base + jp * (2 * CHG)
            handles = []
            for b in range(2):
                pltpu.sync_copy(
                    idx_hbm.at[pl.ds(off + b * CHG, CHG)], idx_v.at[b]
                )
                handles.append(
                    pltpu.async_copy(table_hbm.at[idx_v.at[b]], rows_v.at[b], sems[b])
                )
            for b in range(2):
                handles[b].wait()
                pltpu.sync_copy(rows_v.at[b], out_hbm.at[pl.ds(off + b * CHG, CHG)])
            return carry

        lax.fori_loop(0, npair, body, 0)

    return k(table, idx)


def _mm(x, w, b):
    m, kdim = x.shape
    n2 = w.shape[1]
    nblk = (m + BMM - 1) // BMM

    def body(x_ref, w_ref, b_ref, out_ref):
        out_ref[...] = (
            jnp.dot(x_ref[...], w_ref[...], preferred_element_type=jnp.float32)
            + b_ref[...]
        )

    return pl.pallas_call(
        body,
        grid=(nblk,),
        in_specs=[
            pl.BlockSpec((BMM, kdim), lambda c: (c, 0)),
            pl.BlockSpec((kdim, n2), lambda c: (0, 0)),
            pl.BlockSpec((1, n2), lambda c: (0, 0)),
        ],
        out_specs=pl.BlockSpec((BMM, n2), lambda c: (c, 0)),
        out_shape=jax.ShapeDtypeStruct((m, n2), jnp.float32),
    )(x, w, b.reshape(1, n2))


def _bdmm(ge, wstack, relmap):
    l1, din = ge.shape
    nblk = l1 // BM

    grid_spec = pltpu.PrefetchScalarGridSpec(
        num_scalar_prefetch=1,
        grid=(nblk,),
        in_specs=[
            pl.BlockSpec((BM, din), lambda c, rm: (c, 0)),
            pl.BlockSpec((1, din, H), lambda c, rm: (rm[c], 0, 0)),
        ],
        out_specs=pl.BlockSpec((BM, H), lambda c, rm: (c, 0)),
    )

    def body(rm_ref, ge_ref, w_ref, out_ref):
        out_ref[...] = jnp.dot(
            ge_ref[...], w_ref[0], preferred_element_type=jnp.float32
        )

    return pl.pallas_call(
        body,
        grid_spec=grid_spec,
        out_shape=jax.ShapeDtypeStruct((l1, H), jnp.float32),
    )(relmap, ge, wstack)


def _seg_emp(msg, dl2, sw, blkmap, firstmap, lastmap, nb):
    nch = msg.shape[0] // CH

    grid_spec = pltpu.PrefetchScalarGridSpec(
        num_scalar_prefetch=3,
        grid=(nch,),
        in_specs=[
            pl.BlockSpec((CH, H), lambda c, bm, fm, lm: (c, 0)),
            pl.BlockSpec((1, 1, CH), lambda c, bm, fm, lm: (c, 0, 0)),
            pl.BlockSpec((BO, H), lambda c, bm, fm, lm: (bm[c], 0)),
        ],
        out_specs=pl.BlockSpec((BO, H), lambda c, bm, fm, lm: (bm[c], 0)),
    )

    def body(bm_ref, fm_ref, lm_ref, msg_ref, dl_ref, sw_ref, out_ref):
        c = pl.program_id(0)
        oh = (
            lax.broadcasted_iota(jnp.int32, (BO, CH), 0) == dl_ref[0]
        ).astype(jnp.float32)
        contrib = jnp.dot(oh, msg_ref[...], preferred_element_type=jnp.float32)

        @pl.when(fm_ref[c] == 1)
        def _():
            out_ref[...] = sw_ref[...] + contrib

        @pl.when(fm_ref[c] == 0)
        def _():
            out_ref[...] = out_ref[...] + contrib

        @pl.when(lm_ref[c] == 1)
        def _():
            out_ref[...] = jnp.maximum(out_ref[...], 0.0)

    return pl.pallas_call(
        body,
        grid_spec=grid_spec,
        out_shape=jax.ShapeDtypeStruct((nb * BO, H), jnp.float32),
    )(blkmap, firstmap, lastmap, msg, dl2.reshape(nch, 1, CH), sw)


def _seg_rgcn(ma, mb, dl2, init, blkmap, firstmap, nb):
    nch = ma.shape[0] // CH

    grid_spec = pltpu.PrefetchScalarGridSpec(
        num_scalar_prefetch=2,
        grid=(nch,),
        in_specs=[
            pl.BlockSpec((CH, H), lambda c, bm, fm: (c, 0)),
            pl.BlockSpec((CH, H), lambda c, bm, fm: (c, 0)),
            pl.BlockSpec((1, 1, CH), lambda c, bm, fm: (c, 0, 0)),
            pl.BlockSpec((BO, H), lambda c, bm, fm: (bm[c], 0)),
        ],
        out_specs=pl.BlockSpec((BO, H), lambda c, bm, fm: (bm[c], 0)),
    )

    def body(bm_ref, fm_ref, ma_ref, mb_ref, dl_ref, init_ref, out_ref):
        c = pl.program_id(0)
        oh = (
            lax.broadcasted_iota(jnp.int32, (BO, CH), 0) == dl_ref[0]
        ).astype(jnp.float32)
        contrib = jnp.dot(
            oh, ma_ref[...] + mb_ref[...], preferred_element_type=jnp.float32
        )

        @pl.when(fm_ref[c] == 1)
        def _():
            out_ref[...] = init_ref[...] + contrib

        @pl.when(fm_ref[c] == 0)
        def _():
            out_ref[...] = out_ref[...] + contrib

    return pl.pallas_call(
        body,
        grid_spec=grid_spec,
        out_shape=jax.ShapeDtypeStruct((nb * BO, H), jnp.float32),
    )(blkmap, firstmap, ma, mb, dl2.reshape(nch, 1, CH), init)


def _stats(y):
    n = y.shape[0]
    chs = 256
    nblk = (n + chs - 1) // chs

    def body(y_ref, out_ref):
        c = pl.program_id(0)
        rows = lax.broadcasted_iota(jnp.int32, (chs, 1), 0) + c * chs
        ym = jnp.where(rows < n, y_ref[...], 0.0)

        @pl.when(c == 0)
        def _():
            out_ref[...] = jnp.zeros((8, H), jnp.float32)

        out_ref[0:1, :] = out_ref[0:1, :] + jnp.sum(ym, axis=0, keepdims=True)
        out_ref[1:2, :] = out_ref[1:2, :] + jnp.sum(ym * ym, axis=0, keepdims=True)

    return pl.pallas_call(
        body,
        grid=(nblk,),
        in_specs=[pl.BlockSpec((chs, H), lambda c: (c, 0))],
        out_specs=pl.BlockSpec((8, H), lambda c: (0, 0)),
        out_shape=jax.ShapeDtypeStruct((8, H), jnp.float32),
    )(y)


def _bn_apply(y, st, g, beta, relu, n):
    rows = y.shape[0]
    chb = 512
    nblk = (rows + chb - 1) // chb

    def body(y_ref, s_ref, g_ref, b_ref, out_ref):
        m = s_ref[0:1, :] / n
        v = s_ref[1:2, :] / n - m * m
        inv = lax.rsqrt(v + 1e-5)
        o = (y_ref[...] - m) * inv * g_ref[...] + b_ref[...]
        if relu:
            o = jnp.maximum(o, 0.0)
        out_ref[...] = o

    return pl.pallas_call(
        body,
        grid=(nblk,),
        in_specs=[
            pl.BlockSpec((chb, H), lambda c: (c, 0)),
            pl.BlockSpec((8, H), lambda c: (0, 0)),
            pl.BlockSpec((1, H), lambda c: (0, 0)),
            pl.BlockSpec((1, H), lambda c: (0, 0)),
        ],
        out_specs=pl.BlockSpec((chb, H), lambda c: (c, 0)),
        out_shape=jax.ShapeDtypeStruct((rows, H), jnp.float32),
    )(y, st, g, beta)


def _pool(h, bids2, g):
    nchp = bids2.shape[0]
    n = h.shape[0]

    def body(h_ref, b_ref, sum_ref, cnt_ref):
        c = pl.program_id(0)
        rows = lax.broadcasted_iota(jnp.int32, (CH, 1), 0) + c * CH
        hm = jnp.where(rows < n, h_ref[...], 0.0)
        oh = (
            lax.broadcasted_iota(jnp.int32, (g, CH), 0) == b_ref[0]
        ).astype(jnp.float32)

        @pl.when(c == 0)
        def _():
            sum_ref[...] = jnp.zeros((g, H), jnp.float32)
            cnt_ref[...] = jnp.zeros((g, 128), jnp.float32)

        sum_ref[...] = sum_ref[...] + jnp.dot(
            oh, hm, preferred_element_type=jnp.float32
        )
        cnt_ref[...] = cnt_ref[...] + jnp.dot(
            oh, jnp.ones((CH, 128), jnp.float32), preferred_element_type=jnp.float32
        )

    return pl.pallas_call(
        body,
        grid=(nchp,),
        in_specs=[
            pl.BlockSpec((CH, H), lambda c: (c, 0)),
            pl.BlockSpec((1, 1, CH), lambda c: (c, 0, 0)),
        ],
        out_specs=[
            pl.BlockSpec((g, H), lambda c: (0, 0)),
            pl.BlockSpec((g, 128), lambda c: (0, 0)),
        ],
        out_shape=[
            jax.ShapeDtypeStruct((g, H), jnp.float32),
            jax.ShapeDtypeStruct((g, 128), jnp.float32),
        ],
    )(h, bids2.reshape(nchp, 1, CH))


def _mlp(sums, cnts, p1, b1, p2, b2, p3, b3):
    g = sums.shape[0]
    dout = p3.shape[1]

    def body(s_ref, c_ref, p1_ref, b1_ref, p2_ref, b2_ref, p3_ref, b3_ref, out_ref):
        cnt = jnp.maximum(c_ref[:, 0:1], 1.0)
        pooled = s_ref[...] / cnt
        z = jnp.dot(pooled, p1_ref[...], preferred_element_type=jnp.float32) + b1_ref[...]
        z = jnp.maximum(z, 0.0)
        z = jnp.dot(z, p2_ref[...], preferred_element_type=jnp.float32) + b2_ref[...]
        z = jnp.dot(z, p3_ref[...], preferred_element_type=jnp.float32) + b3_ref[...]
        out_ref[...] = z

    return pl.pallas_call(
        body,
        out_shape=jax.ShapeDtypeStruct((g, dout), jnp.float32),
    )(sums, cnts, p1, b1.reshape(1, -1), p2, b2.reshape(1, -1), p3, b3.reshape(1, -1))


def _seg_layout(dst_ids, num_rows, nb):
    ne = dst_ids.shape[0]
    order = jnp.argsort(dst_ids)
    d_s = dst_ids[order]
    blk_s = d_s // BO
    cnt_b = jnp.bincount(blk_s, length=nb)
    chunks_b = jnp.maximum(1, (cnt_b + CH - 1) // CH)
    bounds = jnp.cumsum(chunks_b)
    cstart = bounds - chunks_b
    ecum = jnp.cumsum(cnt_b) - cnt_b
    pos = jnp.arange(ne, dtype=jnp.int32) + (cstart[blk_s] * CH - ecum[blk_s]).astype(jnp.int32)
    length = _rup(ne + nb * CH, NW * CH)
    nch = length // CH
    dloc = jnp.full((length,), -1, jnp.int32).at[pos].set(
        (d_s - blk_s * BO).astype(jnp.int32))
    blkmap = jnp.minimum(
        jnp.searchsorted(bounds, jnp.arange(nch), side="right"), nb - 1
    ).astype(jnp.int32)
    firstmap = jnp.zeros((nch,), jnp.int32).at[cstart].set(1)
    lastmap = jnp.zeros((nch,), jnp.int32).at[bounds - 1].set(1)
    return order, pos, length, blkmap, firstmap, lastmap, dloc


def kernel(x, edge_attr, params, edge_index, edge_type, edge_message_index,
           edge_message_relation, batch):
    n = x.shape[0]
    e_cnt = edge_index.shape[1]
    e2_cnt = edge_message_index.shape[1]
    g = 32
    p = params

    src = edge_index[0].astype(jnp.int32)
    dst = edge_index[1].astype(jnp.int32)
    esrc = edge_message_index[0].astype(jnp.int32)
    edst = edge_message_index[1].astype(jnp.int32)
    erel = edge_message_relation.astype(jnp.int32)
    etype = edge_type.astype(jnp.int32)

    order1 = jnp.argsort(erel)
    rel_s = erel[order1]
    cnt_r = jnp.bincount(erel, length=R_E)
    pad_r = (cnt_r + BM - 1) // BM * BM - cnt_r
    padoff = (jnp.cumsum(pad_r) - pad_r).astype(jnp.int32)
    pos1 = jnp.arange(e2_cnt, dtype=jnp.int32) + padoff[rel_s]
    l1 = _rup(e2_cnt + R_E * BM, NW * CH)
    gidx1 = jnp.zeros((l1,), jnp.int32).at[pos1].set(esrc[order1])
    relmap = jnp.zeros((l1 // BM,), jnp.int32).at[pos1 // BM].set(rel_s)
    pos1_orig = jnp.zeros((e2_cnt,), jnp.int32).at[order1].set(pos1)

    nbe = (e_cnt + BO - 1) // BO
    order3, pos3, l3, blkmap3, first3, last3, dl3 = _seg_layout(edst, e_cnt, nbe)
    idx2 = jnp.zeros((l3,), jnp.int32).at[pos3].set(pos1_orig[order3])

    nbn = (n + BO - 1) // BO
    order3n, pos3n, l3n, blkmapn, firstn, _lastn, dl3n = _seg_layout(dst, n, nbn)
    idx_a = jnp.zeros((l3n,), jnp.int32).at[pos3n].set(
        (src * R_N + etype)[order3n])
    idx_b = jnp.zeros((l3n,), jnp.int32).at[pos3n].set(order3n)

    nchp = (n + CH - 1) // CH
    bids2 = jnp.full((nchp * CH,), g + 1, jnp.int32).at[
        jnp.arange(n, dtype=jnp.int32)].set(batch.astype(jnp.int32)).reshape(nchp, CH)

    zeros_h = jnp.zeros((1, H), jnp.float32)
    bns = ["bn1", "bn2", "bn3", "bn3", "bn3", "bn2"]

    e = edge_attr
    h = x
    for i in range(6):
        li = i + 1
        w_emp = p["emp%d_W" % li]
        ws = p["emp%d_Ws" % li]
        b_emp = p["emp%d_b" % li]
        din = e.shape[1]
        dpad = _rup(din, 128)
        if dpad != din:
            ep = jnp.pad(e, ((0, 0), (0, dpad - din)))
            wp = jnp.pad(w_emp, ((0, 0), (0, dpad - din), (0, 0)))
            wsp = jnp.pad(ws, ((0, dpad - din), (0, 0)))
        else:
            ep, wp, wsp = e, w_emp, ws

        ge = _sc_gather(ep, gidx1)
        msg1 = _bdmm(ge, wp, relmap)
        msg3 = _sc_gather(msg1, idx2)
        sw = _mm(ep, wsp, b_emp)
        sw_pad = jnp.pad(sw, ((0, nbe * BO - e_cnt), (0, 0)))
        e = _seg_emp(msg3, dl3, sw_pad, blkmap3, first3, last3, nbe)[:e_cnt]

        w_conv = p["conv%d_W" % li]
        wr = p["conv%d_Wr" % li]
        we = p["conv%d_We" % li]
        b_conv = p["conv%d_b" % li]
        dinh = h.shape[1]
        hpad = _rup(dinh, 32)
        if hpad != dinh:
            hp = jnp.pad(h, ((0, 0), (0, hpad - dinh)))
            wcp = jnp.pad(w_conv, ((0, 0), (0, hpad - dinh), (0, 0)))
            wrp = jnp.pad(wr, ((0, hpad - dinh), (0, 0)))
        else:
            hp, wcp, wrp = h, w_conv, wr

        hw2 = jnp.stack(
            [_mm(hp, wcp[r], zeros_h) for r in range(R_N)], axis=1
        ).reshape(R_N * n, H)
        ewe = _mm(e, we, zeros_h)
        ma = _sc_gather(hw2, idx_a)
        mb = _sc_gather(ewe, idx_b)
        hwr = _mm(hp, wrp, b_conv)
        hwr_pad = jnp.pad(hwr, ((0, nbn * BO - n), (0, 0)))
        y = _seg_rgcn(ma, mb, dl3n, hwr_pad, blkmapn, firstn, nbn)[:n]

        st = _stats(y)
        gg = p[bns[i] + "_g"].reshape(1, H)
        bb = p[bns[i] + "_beta"].reshape(1, H)
        h = _bn_apply(y, st, gg, bb, relu=(i < 5), n=float(n))

    sums, cnts = _pool(h, bids2, g)

    dmid = p["P2"].shape[1]
    dpad2 = _rup(dmid, 128)
    p2p = jnp.pad(p["P2"], ((0, 0), (0, dpad2 - dmid)))
    pb2p = jnp.pad(p["pb2"], (0, dpad2 - dmid))
    p3p = jnp.pad(p["P3"], ((0, dpad2 - dmid), (0, dpad2 - dmid)))
    pb3p = jnp.pad(p["pb3"], (0, dpad2 - dmid))
    z = _mlp(sums, cnts, p["P1"], p["pb1"], p2p, pb2p, p3p, pb3p)
    return z[:, :dmid]


# speedup vs baseline: 1.0058x; 1.0032x over previous
import functools

import jax
import jax.numpy as jnp
from jax import lax
from jax.experimental import pallas as pl
from jax.experimental.pallas import tpu as pltpu
from jax.experimental.pallas import tpu_sc as plsc

H = 512
CH = 128
BO = 256
BM = 256
BMM = 512
NW = 32
R_E = 8
R_N = 7


def _rup(x, m):
    return (x + m - 1) // m * m


CHG = 32
NBUF = 4


def _sc_gather(table, idx):
    b_total = idx.shape[0]
    d = table.shape[1]
    bw = b_total // NW
    nrounds = bw // (CHG * NBUF)
    mesh = plsc.VectorSubcoreMesh(core_axis_name="c", subcore_axis_name="s")

    @functools.partial(
        pl.kernel,
        mesh=mesh,
        out_type=jax.ShapeDtypeStruct((b_total, d), jnp.float32),
        scratch_types=[
            pltpu.VMEM((NBUF, CHG), jnp.int32),
            pltpu.VMEM((NBUF, CHG, d), jnp.float32),
        ]
        + [pltpu.SemaphoreType.DMA] * (2 * NBUF),
    )
    def k(table_hbm, idx_hbm, out_hbm, idx_v, rows_v, *sems):
        gsems, osems = sems[:NBUF], sems[NBUF:]
        wid = lax.axis_index("s") * 2 + lax.axis_index("c")
        base = wid * bw

        def body(jp, carry):
            off = base + jp * (NBUF * CHG)
            handles = []
            for b in range(NBUF):
                @pl.when(jp > 0)
                def _(b=b):
                    pltpu.make_async_copy(
                        rows_v.at[b], out_hbm.at[pl.ds(base, CHG)], osems[b]
                    ).wait()

                pltpu.sync_copy(idx_hbm.at[pl.ds(off + b * CHG, CHG)], idx_v.at[b])
                handles.append(
                    pltpu.async_copy(table_hbm.at[idx_v.at[b]], rows_v.at[b], gsems[b])
                )
            for b in range(NBUF):
                handles[b].wait()
                pltpu.async_copy(
                    rows_v.at[b], out_hbm.at[pl.ds(off + b * CHG, CHG)], osems[b]
                )
            return carry

        lax.fori_loop(0, nrounds, body, 0)
        for b in range(NBUF):
            pltpu.make_async_copy(
                rows_v.at[b], out_hbm.at[pl.ds(base, CHG)], osems[b]
            ).wait()

    return k(table, idx)


def _mm(x, w, b):
    m, kdim = x.shape
    n2 = w.shape[1]
    nblk = (m + BMM - 1) // BMM

    def body(x_ref, w_ref, b_ref, out_ref):
        out_ref[...] = (
            jnp.dot(x_ref[...], w_ref[...], preferred_element_type=jnp.float32)
            + b_ref[...]
        )

    return pl.pallas_call(
        body,
        grid=(nblk,),
        in_specs=[
            pl.BlockSpec((BMM, kdim), lambda c: (c, 0)),
            pl.BlockSpec((kdim, n2), lambda c: (0, 0)),
            pl.BlockSpec((1, n2), lambda c: (0, 0)),
        ],
        out_specs=pl.BlockSpec((BMM, n2), lambda c: (c, 0)),
        out_shape=jax.ShapeDtypeStruct((m, n2), jnp.float32),
    )(x, w, b.reshape(1, n2))


def _bdmm(ge, wstack, relmap):
    l1, din = ge.shape
    nblk = l1 // BM

    grid_spec = pltpu.PrefetchScalarGridSpec(
        num_scalar_prefetch=1,
        grid=(nblk,),
        in_specs=[
            pl.BlockSpec((BM, din), lambda c, rm: (c, 0)),
            pl.BlockSpec((1, din, H), lambda c, rm: (rm[c], 0, 0)),
        ],
        out_specs=pl.BlockSpec((BM, H), lambda c, rm: (c, 0)),
    )

    def body(rm_ref, ge_ref, w_ref, out_ref):
        out_ref[...] = jnp.dot(
            ge_ref[...], w_ref[0], preferred_element_type=jnp.float32
        )

    return pl.pallas_call(
        body,
        grid_spec=grid_spec,
        out_shape=jax.ShapeDtypeStruct((l1, H), jnp.float32),
    )(relmap, ge, wstack)


def _seg_emp(msg, dl2, sw, blkmap, firstmap, lastmap, nb):
    nch = msg.shape[0] // CH

    grid_spec = pltpu.PrefetchScalarGridSpec(
        num_scalar_prefetch=3,
        grid=(nch,),
        in_specs=[
            pl.BlockSpec((CH, H), lambda c, bm, fm, lm: (c, 0)),
            pl.BlockSpec((1, 1, CH), lambda c, bm, fm, lm: (c, 0, 0)),
            pl.BlockSpec((BO, H), lambda c, bm, fm, lm: (bm[c], 0)),
        ],
        out_specs=pl.BlockSpec((BO, H), lambda c, bm, fm, lm: (bm[c], 0)),
    )

    def body(bm_ref, fm_ref, lm_ref, msg_ref, dl_ref, sw_ref, out_ref):
        c = pl.program_id(0)
        oh = (
            lax.broadcasted_iota(jnp.int32, (BO, CH), 0) == dl_ref[0]
        ).astype(jnp.float32)
        contrib = jnp.dot(oh, msg_ref[...], preferred_element_type=jnp.float32)

        @pl.when(fm_ref[c] == 1)
        def _():
            out_ref[...] = sw_ref[...] + contrib

        @pl.when(fm_ref[c] == 0)
        def _():
            out_ref[...] = out_ref[...] + contrib

        @pl.when(lm_ref[c] == 1)
        def _():
            out_ref[...] = jnp.maximum(out_ref[...], 0.0)

    return pl.pallas_call(
        body,
        grid_spec=grid_spec,
        out_shape=jax.ShapeDtypeStruct((nb * BO, H), jnp.float32),
    )(blkmap, firstmap, lastmap, msg, dl2.reshape(nch, 1, CH), sw)


def _seg_rgcn(ma, mb, dl2, init, blkmap, firstmap, nb):
    nch = ma.shape[0] // CH

    grid_spec = pltpu.PrefetchScalarGridSpec(
        num_scalar_prefetch=2,
        grid=(nch,),
        in_specs=[
            pl.BlockSpec((CH, H), lambda c, bm, fm: (c, 0)),
            pl.BlockSpec((CH, H), lambda c, bm, fm: (c, 0)),
            pl.BlockSpec((1, 1, CH), lambda c, bm, fm: (c, 0, 0)),
            pl.BlockSpec((BO, H), lambda c, bm, fm: (bm[c], 0)),
        ],
        out_specs=pl.BlockSpec((BO, H), lambda c, bm, fm: (bm[c], 0)),
    )

    def body(bm_ref, fm_ref, ma_ref, mb_ref, dl_ref, init_ref, out_ref):
        c = pl.program_id(0)
        oh = (
            lax.broadcasted_iota(jnp.int32, (BO, CH), 0) == dl_ref[0]
        ).astype(jnp.float32)
        contrib = jnp.dot(
            oh, ma_ref[...] + mb_ref[...], preferred_element_type=jnp.float32
        )

        @pl.when(fm_ref[c] == 1)
        def _():
            out_ref[...] = init_ref[...] + contrib

        @pl.when(fm_ref[c] == 0)
        def _():
            out_ref[...] = out_ref[...] + contrib

    return pl.pallas_call(
        body,
        grid_spec=grid_spec,
        out_shape=jax.ShapeDtypeStruct((nb * BO, H), jnp.float32),
    )(blkmap, firstmap, ma, mb, dl2.reshape(nch, 1, CH), init)


def _stats(y):
    n = y.shape[0]
    chs = 256
    nblk = (n + chs - 1) // chs

    def body(y_ref, out_ref):
        c = pl.program_id(0)
        rows = lax.broadcasted_iota(jnp.int32, (chs, 1), 0) + c * chs
        ym = jnp.where(rows < n, y_ref[...], 0.0)

        @pl.when(c == 0)
        def _():
            out_ref[...] = jnp.zeros((8, H), jnp.float32)

        out_ref[0:1, :] = out_ref[0:1, :] + jnp.sum(ym, axis=0, keepdims=True)
        out_ref[1:2, :] = out_ref[1:2, :] + jnp.sum(ym * ym, axis=0, keepdims=True)

    return pl.pallas_call(
        body,
        grid=(nblk,),
        in_specs=[pl.BlockSpec((chs, H), lambda c: (c, 0))],
        out_specs=pl.BlockSpec((8, H), lambda c: (0, 0)),
        out_shape=jax.ShapeDtypeStruct((8, H), jnp.float32),
    )(y)


def _bn_apply(y, st, g, beta, relu, n):
    rows = y.shape[0]
    chb = 512
    nblk = (rows + chb - 1) // chb

    def body(y_ref, s_ref, g_ref, b_ref, out_ref):
        m = s_ref[0:1, :] / n
        v = s_ref[1:2, :] / n - m * m
        inv = lax.rsqrt(v + 1e-5)
        o = (y_ref[...] - m) * inv * g_ref[...] + b_ref[...]
        if relu:
            o = jnp.maximum(o, 0.0)
        out_ref[...] = o

    return pl.pallas_call(
        body,
        grid=(nblk,),
        in_specs=[
            pl.BlockSpec((chb, H), lambda c: (c, 0)),
            pl.BlockSpec((8, H), lambda c: (0, 0)),
            pl.BlockSpec((1, H), lambda c: (0, 0)),
            pl.BlockSpec((1, H), lambda c: (0, 0)),
        ],
        out_specs=pl.BlockSpec((chb, H), lambda c: (c, 0)),
        out_shape=jax.ShapeDtypeStruct((rows, H), jnp.float32),
    )(y, st, g, beta)


def _pool(h, bids2, g):
    nchp = bids2.shape[0]
    n = h.shape[0]

    def body(h_ref, b_ref, sum_ref, cnt_ref):
        c = pl.program_id(0)
        rows = lax.broadcasted_iota(jnp.int32, (CH, 1), 0) + c * CH
        hm = jnp.where(rows < n, h_ref[...], 0.0)
        oh = (
            lax.broadcasted_iota(jnp.int32, (g, CH), 0) == b_ref[0]
        ).astype(jnp.float32)

        @pl.when(c == 0)
        def _():
            sum_ref[...] = jnp.zeros((g, H), jnp.float32)
            cnt_ref[...] = jnp.zeros((g, 128), jnp.float32)

        sum_ref[...] = sum_ref[...] + jnp.dot(
            oh, hm, preferred_element_type=jnp.float32
        )
        cnt_ref[...] = cnt_ref[...] + jnp.dot(
            oh, jnp.ones((CH, 128), jnp.float32), preferred_element_type=jnp.float32
        )

    return pl.pallas_call(
        body,
        grid=(nchp,),
        in_specs=[
            pl.BlockSpec((CH, H), lambda c: (c, 0)),
            pl.BlockSpec((1, 1, CH), lambda c: (c, 0, 0)),
        ],
        out_specs=[
            pl.BlockSpec((g, H), lambda c: (0, 0)),
            pl.BlockSpec((g, 128), lambda c: (0, 0)),
        ],
        out_shape=[
            jax.ShapeDtypeStruct((g, H), jnp.float32),
            jax.ShapeDtypeStruct((g, 128), jnp.float32),
        ],
    )(h, bids2.reshape(nchp, 1, CH))


def _mlp(sums, cnts, p1, b1, p2, b2, p3, b3):
    g = sums.shape[0]
    dout = p3.shape[1]

    def body(s_ref, c_ref, p1_ref, b1_ref, p2_ref, b2_ref, p3_ref, b3_ref, out_ref):
        cnt = jnp.maximum(c_ref[:, 0:1], 1.0)
        pooled = s_ref[...] / cnt
        z = jnp.dot(pooled, p1_ref[...], preferred_element_type=jnp.float32) + b1_ref[...]
        z = jnp.maximum(z, 0.0)
        z = jnp.dot(z, p2_ref[...], preferred_element_type=jnp.float32) + b2_ref[...]
        z = jnp.dot(z, p3_ref[...], preferred_element_type=jnp.float32) + b3_ref[...]
        out_ref[...] = z

    return pl.pallas_call(
        body,
        out_shape=jax.ShapeDtypeStruct((g, dout), jnp.float32),
    )(sums, cnts, p1, b1.reshape(1, -1), p2, b2.reshape(1, -1), p3, b3.reshape(1, -1))


def _seg_layout(dst_ids, num_rows, nb):
    ne = dst_ids.shape[0]
    order = jnp.argsort(dst_ids)
    d_s = dst_ids[order]
    blk_s = d_s // BO
    cnt_b = jnp.bincount(blk_s, length=nb)
    chunks_b = jnp.maximum(1, (cnt_b + CH - 1) // CH)
    bounds = jnp.cumsum(chunks_b)
    cstart = bounds - chunks_b
    ecum = jnp.cumsum(cnt_b) - cnt_b
    pos = jnp.arange(ne, dtype=jnp.int32) + (cstart[blk_s] * CH - ecum[blk_s]).astype(jnp.int32)
    length = _rup(ne + nb * CH, NW * CH)
    nch = length // CH
    dloc = jnp.full((length,), -1, jnp.int32).at[pos].set(
        (d_s - blk_s * BO).astype(jnp.int32))
    blkmap = jnp.minimum(
        jnp.searchsorted(bounds, jnp.arange(nch), side="right"), nb - 1
    ).astype(jnp.int32)
    firstmap = jnp.zeros((nch,), jnp.int32).at[cstart].set(1)
    lastmap = jnp.zeros((nch,), jnp.int32).at[bounds - 1].set(1)
    return order, pos, length, blkmap, firstmap, lastmap, dloc


def kernel(x, edge_attr, params, edge_index, edge_type, edge_message_index,
           edge_message_relation, batch):
    n = x.shape[0]
    e_cnt = edge_index.shape[1]
    e2_cnt = edge_message_index.shape[1]
    g = 32
    p = params

    src = edge_index[0].astype(jnp.int32)
    dst = edge_index[1].astype(jnp.int32)
    esrc = edge_message_index[0].astype(jnp.int32)
    edst = edge_message_index[1].astype(jnp.int32)
    erel = edge_message_relation.astype(jnp.int32)
    etype = edge_type.astype(jnp.int32)

    order1 = jnp.argsort(erel)
    rel_s = erel[order1]
    cnt_r = jnp.bincount(erel, length=R_E)
    pad_r = (cnt_r + BM - 1) // BM * BM - cnt_r
    padoff = (jnp.cumsum(pad_r) - pad_r).astype(jnp.int32)
    pos1 = jnp.arange(e2_cnt, dtype=jnp.int32) + padoff[rel_s]
    l1 = _rup(e2_cnt + R_E * BM, NW * CH)
    gidx1 = jnp.zeros((l1,), jnp.int32).at[pos1].set(esrc[order1])
    relmap = jnp.zeros((l1 // BM,), jnp.int32).at[pos1 // BM].set(rel_s)
    pos1_orig = jnp.zeros((e2_cnt,), jnp.int32).at[order1].set(pos1)

    nbe = (e_cnt + BO - 1) // BO
    order3, pos3, l3, blkmap3, first3, last3, dl3 = _seg_layout(edst, e_cnt, nbe)
    idx2 = jnp.zeros((l3,), jnp.int32).at[pos3].set(pos1_orig[order3])

    nbn = (n + BO - 1) // BO
    order3n, pos3n, l3n, blkmapn, firstn, _lastn, dl3n = _seg_layout(dst, n, nbn)
    idx_a = jnp.zeros((l3n,), jnp.int32).at[pos3n].set(
        (src * R_N + etype)[order3n])
    idx_b = jnp.zeros((l3n,), jnp.int32).at[pos3n].set(order3n)

    nchp = (n + CH - 1) // CH
    bids2 = jnp.full((nchp * CH,), g + 1, jnp.int32).at[
        jnp.arange(n, dtype=jnp.int32)].set(batch.astype(jnp.int32)).reshape(nchp, CH)

    zeros_h = jnp.zeros((1, H), jnp.float32)
    bns = ["bn1", "bn2", "bn3", "bn3", "bn3", "bn2"]

    e = edge_attr
    h = x
    for i in range(6):
        li = i + 1
        w_emp = p["emp%d_W" % li]
        ws = p["emp%d_Ws" % li]
        b_emp = p["emp%d_b" % li]
        din = e.shape[1]
        dpad = _rup(din, 128)
        if dpad != din:
            ep = jnp.pad(e, ((0, 0), (0, dpad - din)))
            wp = jnp.pad(w_emp, ((0, 0), (0, dpad - din), (0, 0)))
            wsp = jnp.pad(ws, ((0, dpad - din), (0, 0)))
        else:
            ep, wp, wsp = e, w_emp, ws

        ge = _sc_gather(ep, gidx1)
        msg1 = _bdmm(ge, wp, relmap)
        msg3 = _sc_gather(msg1, idx2)
        sw = _mm(ep, wsp, b_emp)
        sw_pad = jnp.pad(sw, ((0, nbe * BO - e_cnt), (0, 0)))
        e = _seg_emp(msg3, dl3, sw_pad, blkmap3, first3, last3, nbe)[:e_cnt]

        w_conv = p["conv%d_W" % li]
        wr = p["conv%d_Wr" % li]
        we = p["conv%d_We" % li]
        b_conv = p["conv%d_b" % li]
        dinh = h.shape[1]
        hpad = _rup(dinh, 32)
        if hpad != dinh:
            hp = jnp.pad(h, ((0, 0), (0, hpad - dinh)))
            wcp = jnp.pad(w_conv, ((0, 0), (0, hpad - dinh), (0, 0)))
            wrp = jnp.pad(wr, ((0, hpad - dinh), (0, 0)))
        else:
            hp, wcp, wrp = h, w_conv, wr

        hw2 = jnp.stack(
            [_mm(hp, wcp[r], zeros_h) for r in range(R_N)], axis=1
        ).reshape(R_N * n, H)
        ewe = _mm(e, we, zeros_h)
        ma = _sc_gather(hw2, idx_a)
        mb = _sc_gather(ewe, idx_b)
        hwr = _mm(hp, wrp, b_conv)
        hwr_pad = jnp.pad(hwr, ((0, nbn * BO - n), (0, 0)))
        y = _seg_rgcn(ma, mb, dl3n, hwr_pad, blkmapn, firstn, nbn)[:n]

        st = _stats(y)
        gg = p[bns[i] + "_g"].reshape(1, H)
        bb = p[bns[i] + "_beta"].reshape(1, H)
        h = _bn_apply(y, st, gg, bb, relu=(i < 5), n=float(n))

    sums, cnts = _pool(h, bids2, g)

    dmid = p["P2"].shape[1]
    dpad2 = _rup(dmid, 128)
    p2p = jnp.pad(p["P2"], ((0, 0), (0, dpad2 - dmid)))
    pb2p = jnp.pad(p["pb2"], (0, dpad2 - dmid))
    p3p = jnp.pad(p["P3"], ((0, dpad2 - dmid), (0, dpad2 - dmid)))
    pb3p = jnp.pad(p["pb3"], (0, dpad2 - dmid))
    z = _mlp(sums, cnts, p["P1"], p["pb1"], p2p, pb2p, p3p, pb3p)
    return z[:, :dmid]
